# baseline (speedup 1.0000x reference)
import jax
import jax.numpy as jnp
from jax.experimental import pallas as pl
from jax.experimental.pallas import tpu as pltpu

N = 50000
DEG = 16
ORIG_IN = 128
IN_F = 128
OUT_HALF = 64
MID = 160

BLK = 2000
NQ = 25
PART = BLK // NQ
EROWS = PART * DEG
SEG = 80


def _part(nf, nh, e, seg, w1a_t, w1a_b, b1a, w1b, b1b, w2a, b2a, w2b, b2b):
    msgs = [
        jnp.dot(seg, e[c * SEG * DEG:(c + 1) * SEG * DEG, :],
                preferred_element_type=jnp.float32)
        for c in range(PART // SEG)
    ]
    msg = jnp.concatenate(msgs, axis=0)
    h1 = (jnp.dot(nf, w1a_t, preferred_element_type=jnp.float32)
          + jnp.dot(nh, w1a_b, preferred_element_type=jnp.float32) + b1a)
    o1 = jnp.tanh(jnp.dot(jnp.maximum(h1, 0.0), w1b,
                          preferred_element_type=jnp.float32) + b1b)
    h2 = jnp.maximum(jnp.dot(msg, w2a, preferred_element_type=jnp.float32) + b2a, 0.0)
    o2 = jnp.tanh(jnp.dot(h2, w2b, preferred_element_type=jnp.float32) + b2b)
    inv = jax.lax.rsqrt(jnp.sum(o1 * o1, axis=1, keepdims=True)
                        + jnp.sum(o2 * o2, axis=1, keepdims=True))
    return o1 * inv, o2 * inv


def _body(nf_ref, nh_ref, *rest):
    e_refs = rest[:NQ]
    (seg_ref, w1a_t_ref, w1a_b_ref, b1a_ref, w1b_ref, b1b_ref, w2a_ref,
     b2a_ref, w2b_ref, b2b_ref, out_ref) = rest[NQ:]
    w = (w1a_t_ref[...], w1a_b_ref[...], b1a_ref[...], w1b_ref[...],
         b1b_ref[...], w2a_ref[...], b2a_ref[...], w2b_ref[...], b2b_ref[...])
    seg = seg_ref[...]
    for k in range(NQ):
        lo = k * PART
        hi = lo + PART
        o1, o2 = _part(nf_ref[lo:hi], nh_ref[lo:hi], e_refs[k][...], seg, *w)
        out_ref[lo:hi, :OUT_HALF] = o1
        out_ref[lo:hi, OUT_HALF:] = o2


@jax.jit
def kernel(node_features, node_hidden_state, edge_hidden_state,
           W1a, b1a, W1b, b1b, W2a, b2a, W2b, b2b):
    w1a_t = W1a[:ORIG_IN]
    w1a_b = W1a[ORIG_IN:]
    e_v = edge_hidden_state.reshape(N * DEG, OUT_HALF)
    seg = (jnp.arange(SEG)[:, None] == (jnp.arange(SEG * DEG)[None, :] // DEG)
           ).astype(jnp.float32)
    grid = N // BLK

    row_spec = lambda w: pl.BlockSpec((BLK, w), lambda i: (i, 0))
    full_spec = lambda r, c: pl.BlockSpec((r, c), lambda i: (0, 0))

    e_specs = [
        pl.BlockSpec((EROWS, OUT_HALF), lambda i, k=k: (NQ * i + k, 0))
        for k in range(NQ)
    ]

    return pl.pallas_call(
        _body,
        grid=(grid,),
        in_specs=[
            row_spec(ORIG_IN),
            row_spec(IN_F),
            *e_specs,
            full_spec(SEG, SEG * DEG),
            full_spec(ORIG_IN, MID),
            full_spec(IN_F, MID),
            full_spec(1, MID),
            full_spec(MID, OUT_HALF),
            full_spec(1, OUT_HALF),
            full_spec(OUT_HALF, OUT_HALF),
            full_spec(1, OUT_HALF),
            full_spec(OUT_HALF, OUT_HALF),
            full_spec(1, OUT_HALF),
        ],
        out_specs=row_spec(2 * OUT_HALF),
        out_shape=jax.ShapeDtypeStruct((N, 2 * OUT_HALF), jnp.float32),
        compiler_params=pltpu.CompilerParams(
            dimension_semantics=("arbitrary",),
        ),
    )(node_features, node_hidden_state, *([e_v] * NQ), seg,
      w1a_t, w1a_b, b1a.reshape(1, MID), W1b, b1b.reshape(1, OUT_HALF),
      W2a, b2a.reshape(1, OUT_HALF), W2b, b2b.reshape(1, OUT_HALF))


# speedup vs baseline: 1.2372x; 1.2372x over previous
import jax
import jax.numpy as jnp
from jax.experimental import pallas as pl
from jax.experimental.pallas import tpu as pltpu

N = 50000
DEG = 16
ORIG_IN = 128
IN_F = 128
OUT_HALF = 64
MID = 160

BLK = 1000
NQ = 5
PART = BLK // NQ
EROWS = PART * DEG
SEG = 40


def _part(nf, nh, e, seg, w1a_t, w1a_b, b1a, w1b, b1b, w2a, b2a, w2b, b2b):
    msgs = [
        jnp.dot(seg, e[c * SEG * DEG:(c + 1) * SEG * DEG, :],
                preferred_element_type=jnp.float32)
        for c in range(PART // SEG)
    ]
    msg = jnp.concatenate(msgs, axis=0)
    h1 = (jnp.dot(nf, w1a_t, preferred_element_type=jnp.float32)
          + jnp.dot(nh, w1a_b, preferred_element_type=jnp.float32) + b1a)
    o1 = jnp.tanh(jnp.dot(jnp.maximum(h1, 0.0), w1b,
                          preferred_element_type=jnp.float32) + b1b)
    h2 = jnp.maximum(jnp.dot(msg, w2a, preferred_element_type=jnp.float32) + b2a, 0.0)
    o2 = jnp.tanh(jnp.dot(h2, w2b, preferred_element_type=jnp.float32) + b2b)
    inv = jax.lax.rsqrt(jnp.sum(o1 * o1, axis=1, keepdims=True)
                        + jnp.sum(o2 * o2, axis=1, keepdims=True))
    return o1 * inv, o2 * inv


def _body(nf_ref, nh_ref, *rest):
    e_refs = rest[:NQ]
    (seg_ref, w1a_t_ref, w1a_b_ref, b1a_ref, w1b_ref, b1b_ref, w2a_ref,
     b2a_ref, w2b_ref, b2b_ref, out_ref) = rest[NQ:]
    w = (w1a_t_ref[...], w1a_b_ref[...], b1a_ref[...], w1b_ref[...],
         b1b_ref[...], w2a_ref[...], b2a_ref[...], w2b_ref[...], b2b_ref[...])
    seg = seg_ref[...]
    for k in range(NQ):
        lo = k * PART
        hi = lo + PART
        o1, o2 = _part(nf_ref[lo:hi], nh_ref[lo:hi], e_refs[k][...], seg, *w)
        out_ref[lo:hi, :OUT_HALF] = o1
        out_ref[lo:hi, OUT_HALF:] = o2


@jax.jit
def kernel(node_features, node_hidden_state, edge_hidden_state,
           W1a, b1a, W1b, b1b, W2a, b2a, W2b, b2b):
    w1a_t = W1a[:ORIG_IN]
    w1a_b = W1a[ORIG_IN:]
    e_v = edge_hidden_state.reshape(N * DEG, OUT_HALF)
    seg = (jnp.arange(SEG)[:, None] == (jnp.arange(SEG * DEG)[None, :] // DEG)
           ).astype(jnp.float32)
    grid = (2, N // (2 * BLK))

    half = N // (2 * BLK)
    row_spec = lambda w: pl.BlockSpec((BLK, w), lambda c, i: (c * half + i, 0))
    full_spec = lambda r, co: pl.BlockSpec((r, co), lambda c, i: (0, 0))

    e_specs = [
        pl.BlockSpec((EROWS, OUT_HALF), lambda c, i, k=k: ((c * half + i) * NQ + k, 0))
        for k in range(NQ)
    ]

    return pl.pallas_call(
        _body,
        grid=grid,
        in_specs=[
            row_spec(ORIG_IN),
            row_spec(IN_F),
            *e_specs,
            full_spec(SEG, SEG * DEG),
            full_spec(ORIG_IN, MID),
            full_spec(IN_F, MID),
            full_spec(1, MID),
            full_spec(MID, OUT_HALF),
            full_spec(1, OUT_HALF),
            full_spec(OUT_HALF, OUT_HALF),
            full_spec(1, OUT_HALF),
            full_spec(OUT_HALF, OUT_HALF),
            full_spec(1, OUT_HALF),
        ],
        out_specs=row_spec(2 * OUT_HALF),
        out_shape=jax.ShapeDtypeStruct((N, 2 * OUT_HALF), jnp.float32),
        compiler_params=pltpu.CompilerParams(
            dimension_semantics=("parallel", "arbitrary"),
        ),
    )(node_features, node_hidden_state, *([e_v] * NQ), seg,
      w1a_t, w1a_b, b1a.reshape(1, MID), W1b, b1b.reshape(1, OUT_HALF),
      W2a, b2a.reshape(1, OUT_HALF), W2b, b2b.reshape(1, OUT_HALF))


# speedup vs baseline: 1.5233x; 1.2313x over previous
import jax
import jax.numpy as jnp
from jax.experimental import pallas as pl
from jax.experimental.pallas import tpu as pltpu

N = 50000
DEG = 16
ORIG_IN = 128
IN_F = 128
OUT_HALF = 64
MID = 160

BLK = 2000
H = BLK // 2


def _half(nf, nh, e, w1a_t, w1a_b, b1a, w1b, b1b, w2a2, b2a, w2b, b2b):
    s = e[:, :512] + e[:, 512:]
    s = s[:, :256] + s[:, 256:]
    m128 = s[:, :128] + s[:, 128:]
    h1 = (jnp.dot(nf, w1a_t, preferred_element_type=jnp.float32)
          + jnp.dot(nh, w1a_b, preferred_element_type=jnp.float32) + b1a)
    o1 = jnp.tanh(jnp.dot(jnp.maximum(h1, 0.0), w1b,
                          preferred_element_type=jnp.float32) + b1b)
    h2 = jnp.maximum(jnp.dot(m128, w2a2, preferred_element_type=jnp.float32) + b2a, 0.0)
    o2 = jnp.tanh(jnp.dot(h2, w2b, preferred_element_type=jnp.float32) + b2b)
    inv = jax.lax.rsqrt(jnp.sum(o1 * o1, axis=1, keepdims=True)
                        + jnp.sum(o2 * o2, axis=1, keepdims=True))
    return o1 * inv, o2 * inv


def _body(nf_ref, nh_ref, ea_ref, eb_ref, w1a_t_ref, w1a_b_ref, b1a_ref,
          w1b_ref, b1b_ref, w2a2_ref, b2a_ref, w2b_ref, b2b_ref, out_ref):
    w = (w1a_t_ref[...], w1a_b_ref[...], b1a_ref[...], w1b_ref[...],
         b1b_ref[...], w2a2_ref[...], b2a_ref[...], w2b_ref[...], b2b_ref[...])
    oa1, oa2 = _half(nf_ref[:H], nh_ref[:H], ea_ref[...], *w)
    out_ref[:H, :OUT_HALF] = oa1
    out_ref[:H, OUT_HALF:] = oa2
    ob1, ob2 = _half(nf_ref[H:], nh_ref[H:], eb_ref[...], *w)
    out_ref[H:, :OUT_HALF] = ob1
    out_ref[H:, OUT_HALF:] = ob2


@jax.jit
def kernel(node_features, node_hidden_state, edge_hidden_state,
           W1a, b1a, W1b, b1b, W2a, b2a, W2b, b2b):
    w1a_t = W1a[:ORIG_IN]
    w1a_b = W1a[ORIG_IN:]
    e_flat = edge_hidden_state.reshape(N, DEG * OUT_HALF)
    w2a2 = jnp.concatenate([W2a, W2a], axis=0)
    grid = N // BLK

    row_spec = lambda w: pl.BlockSpec((BLK, w), lambda i: (i, 0))
    full_spec = lambda r, c: pl.BlockSpec((r, c), lambda i: (0, 0))

    return pl.pallas_call(
        _body,
        grid=(grid,),
        in_specs=[
            row_spec(ORIG_IN),
            row_spec(IN_F),
            pl.BlockSpec((H, DEG * OUT_HALF), lambda i: (2 * i, 0)),
            pl.BlockSpec((H, DEG * OUT_HALF), lambda i: (2 * i + 1, 0)),
            full_spec(ORIG_IN, MID),
            full_spec(IN_F, MID),
            full_spec(1, MID),
            full_spec(MID, OUT_HALF),
            full_spec(1, OUT_HALF),
            full_spec(2 * OUT_HALF, OUT_HALF),
            full_spec(1, OUT_HALF),
            full_spec(OUT_HALF, OUT_HALF),
            full_spec(1, OUT_HALF),
        ],
        out_specs=row_spec(2 * OUT_HALF),
        out_shape=jax.ShapeDtypeStruct((N, 2 * OUT_HALF), jnp.float32),
        compiler_params=pltpu.CompilerParams(
            dimension_semantics=("arbitrary",),
        ),
    )(node_features, node_hidden_state, e_flat, e_flat,
      w1a_t, w1a_b, b1a.reshape(1, MID), W1b, b1b.reshape(1, OUT_HALF),
      w2a2, b2a.reshape(1, OUT_HALF), W2b, b2b.reshape(1, OUT_HALF))
